# Initial kernel scaffold; baseline (speedup 1.0000x reference)
import jax
import jax.numpy as jnp
from jax.experimental import pallas as pl


def kernel(proposals, gt_labels, gt_bboxes, gt_masks):
    raise NotImplementedError("write your pallas kernel here")


# speedup vs baseline: 1.4281x; 1.4281x over previous
import functools

import jax
import jax.numpy as jnp
from jax import lax
from jax.experimental import pallas as pl
from jax.experimental.pallas import tpu as pltpu
from jax.experimental.pallas import tpu_sc as plsc

_B, _N, _G, _S = 2, 20000, 64, 256
_HI = 0.5
_NFG, _NBG = 32, 96
_MASK = 28
_NTILE = 16
_CHUNK = 1280
_NV = _CHUNK // 16
_NPAD = _NTILE * _CHUNK

_F32 = jnp.float32
_I32 = jnp.int32
_NEG = float("-inf")
_IMAX = 2**31 - 1


def _lane():
    return jnp.arange(16, dtype=_I32)


def _splat(x, dtype):
    return jnp.broadcast_to(jnp.asarray(x, dtype), (16,))


def _bc1(ref, i):
    return plsc.load_gather(ref, [_splat(i, _I32)])


def _bc2(ref, i, j):
    return plsc.load_gather(ref, [_splat(i, _I32), _splat(j, _I32)])


def _log16(x):
    i = plsc.bitcast(x, _I32)
    e = ((i >> 23) & 0xFF) - 127
    m = plsc.bitcast((i & 0x7FFFFF) | 0x3F800000, _F32)
    big = m > 1.4142135
    m = jnp.where(big, m * 0.5, m)
    e = e + big.astype(_I32)
    s = (m - 1.0) / (m + 1.0)
    s2 = s * s
    p = s2 * (1.0 / 9.0)
    p = s2 * (p + 1.0 / 7.0)
    p = s2 * (p + 1.0 / 5.0)
    p = s2 * (p + 1.0 / 3.0)
    p = (p + 1.0) * (2.0 * s)
    return e.astype(_F32) * 0.6931471805599453 + p


def _sc_body(
    pt_hbm,
    gt_hbm,
    lab_hbm,
    mrow_hbm,
    p_o,
    l_o,
    r_o,
    m_o,
    px1_r, py1_r, px2_r, py2_r,
    gx1_r, gy1_r, gx2_r, gy2_r, gar_r,
    lab_r,
    kfg_r, kbg_r,
    gti_r,
    fgv_l, bgv_l,
    fgi_l, fgg_l, bgi_l,
    fgv_s, fgi_s, fgg_s,
    bgv_s, bgi_s,
    keep_s,
    keepg_s,
    inter_s,
    pcon_s,
    fgv_m, fgi_m, fgg_m,
    bgv_m, bgi_m,
    pcon_m,
    pout4_r,
    rout4_r,
    keep_r,
    pout_r,
    lout_r,
    rout_r,
    keepg_r,
    inter_r,
    ridx_r,
    mrows_r,
    x0_r, x1i_r,
    wx_r, wy_r,
    mout_r,
    sem,
):
    b = lax.axis_index("c")
    t = lax.axis_index("s")
    lane = _lane()
    lanef = lane.astype(_F32)

    pltpu.sync_copy(pt_hbm.at[b, 0, pl.ds(t * _CHUNK, _CHUNK)], px1_r)
    pltpu.sync_copy(pt_hbm.at[b, 1, pl.ds(t * _CHUNK, _CHUNK)], py1_r)
    pltpu.sync_copy(pt_hbm.at[b, 2, pl.ds(t * _CHUNK, _CHUNK)], px2_r)
    pltpu.sync_copy(pt_hbm.at[b, 3, pl.ds(t * _CHUNK, _CHUNK)], py2_r)
    pltpu.sync_copy(gt_hbm.at[b, 0], gx1_r)
    pltpu.sync_copy(gt_hbm.at[b, 1], gy1_r)
    pltpu.sync_copy(gt_hbm.at[b, 2], gx2_r)
    pltpu.sync_copy(gt_hbm.at[b, 3], gy2_r)
    pltpu.sync_copy(lab_hbm.at[b], lab_r)

    for c in range(_G // 16):
        sl = pl.ds(c * 16, 16)
        gar_r[sl] = (gx2_r[sl] - gx1_r[sl]) * (gy2_r[sl] - gy1_r[sl])

    def chunk_body(c, _):
        sl = pl.ds(c * 16, 16)
        px1 = px1_r[sl]
        py1 = py1_r[sl]
        px2 = px2_r[sl]
        py2 = py2_r[sl]
        area_p = (px2 - px1) * (py2 - py1)

        def gt_body(g, carry):
            bv, bg = carry
            gx1 = _bc1(gx1_r, g)
            gy1 = _bc1(gy1_r, g)
            gx2 = _bc1(gx2_r, g)
            gy2 = _bc1(gy2_r, g)
            gab = _bc1(gar_r, g)
            wx = jnp.maximum(jnp.minimum(px2, gx2) - jnp.maximum(px1, gx1), 0.0)
            wy = jnp.maximum(jnp.minimum(py2, gy2) - jnp.maximum(py1, gy1), 0.0)
            inter = wx * wy
            den = area_p + gab - inter + 1e-9
            iou = inter / den
            upd = iou > bv
            bv = jnp.where(upd, iou, bv)
            bg = jnp.where(upd, g, bg)
            return bv, bg

        bv, bg = lax.fori_loop(
            0, _G, gt_body,
            (jnp.full((16,), -1.0, _F32), jnp.zeros((16,), _I32)),
        )
        pidx = t * _CHUNK + c * 16 + lane
        valid = pidx < _N
        fg = bv >= _HI
        kfg_r[sl] = jnp.where(fg & valid, bv, _NEG)
        kbg_r[sl] = jnp.where((~fg) & valid, -bv, _NEG)
        gti_r[sl] = bg
        return 0

    lax.fori_loop(0, _NV, chunk_body, 0)

    tbase = t * _CHUNK

    def extract_top(scan_all, K, val_fn, gt_fn):
        nl = K // 16

        def body(k, carry):
            vlast, ilast, lv, li, lg = carry
            bv, bi, bs = scan_all(vlast, ilast)
            vmax = jnp.max(bv)
            imin = jnp.min(jnp.where(bv == vmax, bi, _IMAX))
            slot = jnp.min(jnp.where((bv == vmax) & (bi == imin), bs, _IMAX))
            wm = lane == (k % 16)
            j = k // 16
            valv = val_fn(slot)
            iminv = _splat(imin, _I32)
            lv = tuple(jnp.where(wm & (j == jj), valv, lv[jj])
                       for jj in range(nl))
            li = tuple(jnp.where(wm & (j == jj), iminv, li[jj])
                       for jj in range(nl))
            if gt_fn is not None:
                gv = gt_fn(slot)
                lg = tuple(jnp.where(wm & (j == jj), gv, lg[jj])
                           for jj in range(nl))
            return valv, iminv, lv, li, lg

        zf = tuple(jnp.zeros((16,), _F32) for _ in range(nl))
        zi = tuple(jnp.zeros((16,), _I32) for _ in range(nl))
        zg = zi if gt_fn is not None else ()
        _, _, lv, li, lg = lax.fori_loop(
            0, K, body,
            (jnp.full((16,), jnp.inf, _F32), jnp.full((16,), -1, _I32),
             zf, zi, zg))
        return lv, li, lg

    def scan_init():
        return (jnp.full((16,), _NEG, _F32),
                jnp.full((16,), _IMAX, _I32),
                jnp.zeros((16,), _I32))

    def make_local_scan(keys_ref):
        def scan_all(vlast, ilast):
            def scan(v, inner):
                bv, bi, bs = inner
                kv = keys_ref[pl.ds(v * 16, 16)]
                slot = v * 16 + lane
                iv = tbase + slot
                elig = (kv < vlast) | ((kv == vlast) & (iv > ilast))
                m = elig & ((kv > bv) | ((kv == bv) & (iv < bi)))
                return (
                    jnp.where(m, kv, bv),
                    jnp.where(m, iv, bi),
                    jnp.where(m, slot, bs),
                )

            return lax.fori_loop(0, _NV, scan, scan_init())

        return scan_all

    lv, li, lg = extract_top(
        make_local_scan(kfg_r), _NFG,
        lambda slot: plsc.load_gather(kfg_r, [_splat(slot, _I32)]),
        lambda slot: plsc.load_gather(gti_r, [_splat(slot, _I32)]))
    for jj in range(_NFG // 16):
        sl = pl.ds(jj * 16, 16)
        fgv_l[sl] = lv[jj]
        fgi_l[sl] = li[jj]
        fgg_l[sl] = lg[jj]
    lvb, lib, _ = extract_top(
        make_local_scan(kbg_r), _NBG,
        lambda slot: plsc.load_gather(kbg_r, [_splat(slot, _I32)]), None)
    for jj in range(_NBG // 16):
        sl = pl.ds(jj * 16, 16)
        bgv_l[sl] = lvb[jj]
        bgi_l[sl] = lib[jj]

    pltpu.sync_copy(fgv_l, fgv_s.at[t])
    pltpu.sync_copy(fgi_l, fgi_s.at[t])
    pltpu.sync_copy(fgg_l, fgg_s.at[t])
    pltpu.sync_copy(bgv_l, bgv_s.at[t])
    pltpu.sync_copy(bgi_l, bgi_s.at[t])
    plsc.subcore_barrier()

    @pl.when(t == 0)
    def _phase2():
        pltpu.sync_copy(fgv_s, fgv_m)
        pltpu.sync_copy(fgi_s, fgi_m)
        pltpu.sync_copy(fgg_s, fgg_m)
        pltpu.sync_copy(bgv_s, bgv_m)
        pltpu.sync_copy(bgi_s, bgi_m)

        def make_merge_scan(vals_m, idx_m, ncol):
            cpr = ncol // 16

            def scan_all(vlast, ilast):
                st = scan_init()
                for tt in range(_NTILE):
                    def scan(v, inner, tt=tt):
                        bv, bi, bs = inner
                        kv = vals_m[tt, pl.ds(v * 16, 16)]
                        iv = idx_m[tt, pl.ds(v * 16, 16)]
                        slot = (tt * cpr + v) * 16 + lane
                        elig = (kv < vlast) | ((kv == vlast) & (iv > ilast))
                        m = elig & ((kv > bv) | ((kv == bv) & (iv < bi)))
                        return (
                            jnp.where(m, kv, bv),
                            jnp.where(m, iv, bi),
                            jnp.where(m, slot, bs),
                        )

                    st = lax.fori_loop(0, cpr, scan, st)
                return st

            return scan_all

        mlv, mli, mlg = extract_top(
            make_merge_scan(fgv_m, fgi_m, _NFG), _NFG,
            lambda slot: _bc2(fgv_m, slot // _NFG, slot % _NFG),
            lambda slot: _bc2(fgg_m, slot // _NFG, slot % _NFG))
        for jj in range(_NFG // 16):
            sl = pl.ds(jj * 16, 16)
            keep_r[sl] = mli[jj]
            keepg_r[sl] = mlg[jj]
        blv, bli, _ = extract_top(
            make_merge_scan(bgv_m, bgi_m, _NBG), _NBG,
            lambda slot: _bc2(bgv_m, slot // _NBG, slot % _NBG), None)
        for jj in range(_NBG // 16):
            keep_r[pl.ds(_NFG + jj * 16, 16)] = bli[jj]

        pltpu.sync_copy(keep_r, keep_s)
        pltpu.sync_copy(keepg_r, keepg_s)

    plsc.subcore_barrier()

    pltpu.sync_copy(keep_s, keep_r)
    coord_refs = (px1_r, py1_r, px2_r, py2_r)
    for ch in range(8):
        idxv = keep_r[pl.ds(ch * 16, 16)]
        mine = (idxv >= tbase) & (idxv < tbase + _CHUNK)
        local = jnp.clip(idxv - tbase, 0, _CHUNK - 1)
        for c4 in range(4):
            vals = plsc.load_gather(coord_refs[c4], [local])
            pout4_r[c4, pl.ds(ch * 16, 16)] = jnp.where(mine, vals, 0.0)
    pltpu.sync_copy(pout4_r, pcon_s.at[t])
    plsc.subcore_barrier()

    @pl.when(t == 0)
    def _phase2c():
        pltpu.sync_copy(pcon_s, pcon_m)
        for c4 in range(4):
            for ch in range(8):
                sl = pl.ds(ch * 16, 16)
                acc = pcon_m[0, c4, sl]
                for ti in range(1, _NTILE):
                    acc = acc + pcon_m[ti, c4, sl]
                pout4_r[c4, sl] = acc
        for ch in range(32):
            p = ch * 16 + lane
            vals = plsc.load_gather(pout4_r, [p % 4, p // 4])
            pout_r[pl.ds(ch * 16, 16)] = vals
        pltpu.sync_copy(pout_r, p_o.at[b])

        for ch in range(2):
            gv = keepg_r[pl.ds(ch * 16, 16)]
            labv = plsc.load_gather(lab_r, [gv])
            lout_r[pl.ds(ch * 16, 16)] = labv
        zz = jnp.zeros((16,), _I32)
        for ch in range(6):
            lout_r[pl.ds(_NFG + ch * 16, 16)] = zz
        pltpu.sync_copy(lout_r, l_o.at[b])

        for ch in range(2):
            sl = pl.ds(ch * 16, 16)
            px1 = pout4_r[0, sl]
            py1 = pout4_r[1, sl]
            px2 = pout4_r[2, sl]
            py2 = pout4_r[3, sl]
            gv = keepg_r[sl]
            gx1 = plsc.load_gather(gx1_r, [gv])
            gy1 = plsc.load_gather(gy1_r, [gv])
            gx2 = plsc.load_gather(gx2_r, [gv])
            gy2 = plsc.load_gather(gy2_r, [gv])
            pw = px2 - px1
            ph = py2 - py1
            pxc = px1 + 0.5 * pw
            pyc = py1 + 0.5 * ph
            gw = gx2 - gx1
            gh = gy2 - gy1
            gxc = gx1 + 0.5 * gw
            gyc = gy1 + 0.5 * gh
            regs = (
                (gxc - pxc) / pw,
                (gyc - pyc) / ph,
                _log16(gw / pw),
                _log16(gh / ph),
            )
            for c4 in range(4):
                rout4_r[c4, sl] = regs[c4]
            ix1 = jnp.maximum(px1, gx1)
            iy1 = jnp.maximum(py1, gy1)
            ix2 = jnp.maximum(jnp.minimum(px2, gx2), ix1 + 1e-3)
            iy2 = jnp.maximum(jnp.minimum(py2, gy2), iy1 + 1e-3)
            inter_r[0, sl] = ix1
            inter_r[1, sl] = iy1
            inter_r[2, sl] = ix2
            inter_r[3, sl] = iy2
        for ch in range(8):
            p = ch * 16 + lane
            vals = plsc.load_gather(rout4_r, [p % 4, p // 4])
            rout_r[pl.ds(ch * 16, 16)] = vals
        pltpu.sync_copy(rout_r, r_o.at[b])
        pltpu.sync_copy(inter_r, inter_s)

    plsc.subcore_barrier()

    pltpu.sync_copy(inter_s, inter_r)
    pltpu.sync_copy(keepg_s, keepg_r)
    for so in range(2):
        s = t * 2 + so
        mgt = jnp.clip(plsc.load_gather(keepg_r, [_splat(s, _I32)]), 0, _G - 1)
        base_row = (b * _G + mgt) * _S
        x1 = _bc2(inter_r, 0, s)
        y1 = _bc2(inter_r, 1, s)
        x2 = _bc2(inter_r, 2, s)
        y2 = _bc2(inter_r, 3, s)
        bh = (y2 - y1) / float(_MASK)
        bw = (x2 - x1) / float(_MASK)
        for ch in range(2):
            iv = (lanef + (ch * 16)) + 0.5
            ys = y1 + iv * bh - 0.5
            ysc = jnp.clip(ys, 0.0, float(_S - 1))
            yf = ysc.astype(_I32)
            ycl = jnp.minimum(yf + 1, _S - 1)
            xs = x1 + iv * bw - 0.5
            xsc = jnp.clip(xs, 0.0, float(_S - 1))
            xf = xsc.astype(_I32)
            xcl = jnp.minimum(xf + 1, _S - 1)
            sl = pl.ds(ch * 16, 16)
            ridx_r[sl] = base_row + yf
            ridx_r[pl.ds(32 + ch * 16, 16)] = base_row + ycl
            x0_r[sl] = xf
            x1i_r[sl] = xcl
            wx_r[sl] = xsc - xf.astype(_F32)
            wy_r[sl] = ysc - yf.astype(_F32)
        pltpu.async_copy(mrow_hbm.at[ridx_r], mrows_r, sem).wait()

        def row_body(i, _):
            wyb = _bc1(wy_r, i)
            omy = 1.0 - wyb
            r0 = i
            r1 = 32 + i
            for ch in range(2):
                sl = pl.ds(ch * 16, 16)
                x0v = x0_r[sl]
                x1v = x1i_r[sl]
                wxv = wx_r[sl]
                omx = 1.0 - wxv
                v00 = plsc.load_gather(mrows_r, [_splat(r0, _I32), x0v])
                v01 = plsc.load_gather(mrows_r, [_splat(r0, _I32), x1v])
                v10 = plsc.load_gather(mrows_r, [_splat(r1, _I32), x0v])
                v11 = plsc.load_gather(mrows_r, [_splat(r1, _I32), x1v])
                mf = (v00 * omy * omx + v01 * omy * wxv
                      + v10 * wyb * omx + v11 * wyb * wxv)
                outv = (mf > 0.5).astype(_I32)
                mout_r[pl.ds(i * _MASK + ch * 16, 16)] = outv
            return 0

        lax.fori_loop(0, _MASK, row_body, 0)
        pltpu.sync_copy(mout_r, m_o.at[b, s])


def kernel(proposals, gt_labels, gt_bboxes, gt_masks):
    props_t = jnp.moveaxis(proposals, -1, 1)
    props_t = jnp.pad(props_t, ((0, 0), (0, 0), (0, _NPAD - _N)),
                      constant_values=1.0)
    gt_t = jnp.moveaxis(gt_bboxes, -1, 1)
    labels = gt_labels.astype(_I32)
    mrows = gt_masks.reshape(_B * _G * _S, _S)

    mesh = plsc.VectorSubcoreMesh(core_axis_name="c", subcore_axis_name="s")
    call = pl.kernel(
        _sc_body,
        out_type=(
            jax.ShapeDtypeStruct((_B, 512), _F32),
            jax.ShapeDtypeStruct((_B, 128), _I32),
            jax.ShapeDtypeStruct((_B, 128), _F32),
            jax.ShapeDtypeStruct((_B, 32, 800), _I32),
        ),
        mesh=mesh,
        compiler_params=pltpu.CompilerParams(needs_layout_passes=False),
        scratch_types=[
            pltpu.VMEM((_CHUNK,), _F32),
            pltpu.VMEM((_CHUNK,), _F32),
            pltpu.VMEM((_CHUNK,), _F32),
            pltpu.VMEM((_CHUNK,), _F32),
            pltpu.VMEM((_G,), _F32),
            pltpu.VMEM((_G,), _F32),
            pltpu.VMEM((_G,), _F32),
            pltpu.VMEM((_G,), _F32),
            pltpu.VMEM((_G,), _F32),
            pltpu.VMEM((_G,), _I32),
            pltpu.VMEM((_CHUNK,), _F32),
            pltpu.VMEM((_CHUNK,), _F32),
            pltpu.VMEM((_CHUNK,), _I32),
            pltpu.VMEM((128,), _F32),
            pltpu.VMEM((128,), _F32),
            pltpu.VMEM((128,), _I32),
            pltpu.VMEM((128,), _I32),
            pltpu.VMEM((128,), _I32),
            pltpu.VMEM_SHARED((_NTILE, 128), _F32),
            pltpu.VMEM_SHARED((_NTILE, 128), _I32),
            pltpu.VMEM_SHARED((_NTILE, 128), _I32),
            pltpu.VMEM_SHARED((_NTILE, 128), _F32),
            pltpu.VMEM_SHARED((_NTILE, 128), _I32),
            pltpu.VMEM_SHARED((128,), _I32),
            pltpu.VMEM_SHARED((_NFG,), _I32),
            pltpu.VMEM_SHARED((4, _NFG), _F32),
            pltpu.VMEM_SHARED((_NTILE, 4, 128), _F32),
            pltpu.VMEM((_NTILE, 128), _F32),
            pltpu.VMEM((_NTILE, 128), _I32),
            pltpu.VMEM((_NTILE, 128), _I32),
            pltpu.VMEM((_NTILE, 128), _F32),
            pltpu.VMEM((_NTILE, 128), _I32),
            pltpu.VMEM((_NTILE, 4, 128), _F32),
            pltpu.VMEM((4, 128), _F32),
            pltpu.VMEM((4, _NFG), _F32),
            pltpu.VMEM((128,), _I32),
            pltpu.VMEM((512,), _F32),
            pltpu.VMEM((128,), _I32),
            pltpu.VMEM((128,), _F32),
            pltpu.VMEM((_NFG,), _I32),
            pltpu.VMEM((4, _NFG), _F32),
            pltpu.VMEM((64,), _I32),
            pltpu.VMEM((64, _S), _F32),
            pltpu.VMEM((_NFG,), _I32),
            pltpu.VMEM((_NFG,), _I32),
            pltpu.VMEM((_NFG,), _F32),
            pltpu.VMEM((_NFG,), _F32),
            pltpu.VMEM((800,), _I32),
            pltpu.SemaphoreType.DMA,
        ],
    )
    p, l, r, mk = call(props_t, gt_t, labels, mrows)
    return (
        p.reshape(_B, 128, 4),
        l,
        r.reshape(_B, 32, 4),
        mk[:, :, : _MASK * _MASK].reshape(_B, 32, 28, 28),
    )


# speedup vs baseline: 2.1790x; 1.5258x over previous
import functools

import jax
import jax.numpy as jnp
from jax import lax
from jax.experimental import pallas as pl
from jax.experimental.pallas import tpu as pltpu
from jax.experimental.pallas import tpu_sc as plsc

_B, _N, _G, _S = 2, 20000, 64, 256
_HI = 0.5
_NFG, _NBG = 32, 96
_MASK = 28
_NTILE = 16
_CHUNK = 1280
_NV = _CHUNK // 16
_NPAD = _NTILE * _CHUNK

_F32 = jnp.float32
_I32 = jnp.int32
_NEG = float("-inf")
_IMAX = 2**31 - 1


def _lane():
    return jnp.arange(16, dtype=_I32)


def _splat(x, dtype):
    return jnp.broadcast_to(jnp.asarray(x, dtype), (16,))


def _bc1(ref, i):
    return plsc.load_gather(ref, [_splat(i, _I32)])


def _bc2(ref, i, j):
    return plsc.load_gather(ref, [_splat(i, _I32), _splat(j, _I32)])


def _log16(x):
    i = plsc.bitcast(x, _I32)
    e = ((i >> 23) & 0xFF) - 127
    m = plsc.bitcast((i & 0x7FFFFF) | 0x3F800000, _F32)
    big = m > 1.4142135
    m = jnp.where(big, m * 0.5, m)
    e = e + big.astype(_I32)
    s = (m - 1.0) / (m + 1.0)
    s2 = s * s
    p = s2 * (1.0 / 9.0)
    p = s2 * (p + 1.0 / 7.0)
    p = s2 * (p + 1.0 / 5.0)
    p = s2 * (p + 1.0 / 3.0)
    p = (p + 1.0) * (2.0 * s)
    return e.astype(_F32) * 0.6931471805599453 + p


def _sc_body(
    pt_hbm,
    gt_hbm,
    lab_hbm,
    mrow_hbm,
    p_o,
    l_o,
    r_o,
    m_o,
    px1_r, py1_r, px2_r, py2_r,
    gx1_r, gy1_r, gx2_r, gy2_r, gar_r,
    lab_r,
    kfg_r, kbg_r,
    gti_r,
    fgv_l, bgv_l,
    fgi_l, fgg_l, bgi_l,
    fgv_s, fgi_s, fgg_s,
    bgv_s, bgi_s,
    keep_s,
    keepg_s,
    inter_s,
    pcon_s,
    fgv_m, fgi_m, fgg_m,
    bgv_m, bgi_m,
    pcon_m,
    pout4_r,
    rout4_r,
    keep_r,
    pout_r,
    lout_r,
    rout_r,
    keepg_r,
    inter_r,
    ridx_r,
    mrows_r,
    x0_r, x1i_r,
    wx_r, wy_r,
    mout_r,
    sem,
):
    b = lax.axis_index("c")
    t = lax.axis_index("s")
    lane = _lane()
    lanef = lane.astype(_F32)

    pltpu.sync_copy(pt_hbm.at[b, 0, pl.ds(t * _CHUNK, _CHUNK)], px1_r)
    pltpu.sync_copy(pt_hbm.at[b, 1, pl.ds(t * _CHUNK, _CHUNK)], py1_r)
    pltpu.sync_copy(pt_hbm.at[b, 2, pl.ds(t * _CHUNK, _CHUNK)], px2_r)
    pltpu.sync_copy(pt_hbm.at[b, 3, pl.ds(t * _CHUNK, _CHUNK)], py2_r)
    pltpu.sync_copy(gt_hbm.at[b, 0], gx1_r)
    pltpu.sync_copy(gt_hbm.at[b, 1], gy1_r)
    pltpu.sync_copy(gt_hbm.at[b, 2], gx2_r)
    pltpu.sync_copy(gt_hbm.at[b, 3], gy2_r)
    pltpu.sync_copy(lab_hbm.at[b], lab_r)

    for c in range(_G // 16):
        sl = pl.ds(c * 16, 16)
        gar_r[sl] = (gx2_r[sl] - gx1_r[sl]) * (gy2_r[sl] - gy1_r[sl])

    _QB = 4

    def chunk_body(cb, _):
        sls = [pl.ds((cb * _QB + q) * 16, 16) for q in range(_QB)]
        px1 = [px1_r[s] for s in sls]
        py1 = [py1_r[s] for s in sls]
        px2 = [px2_r[s] for s in sls]
        py2 = [py2_r[s] for s in sls]
        area_p = [(px2[q] - px1[q]) * (py2[q] - py1[q]) for q in range(_QB)]

        def gt_body(g, carry):
            bvs, bgs = carry
            gx1 = _bc1(gx1_r, g)
            gy1 = _bc1(gy1_r, g)
            gx2 = _bc1(gx2_r, g)
            gy2 = _bc1(gy2_r, g)
            gab = _bc1(gar_r, g)
            nbv, nbg = [], []
            for q in range(_QB):
                wx = jnp.maximum(
                    jnp.minimum(px2[q], gx2) - jnp.maximum(px1[q], gx1), 0.0)
                wy = jnp.maximum(
                    jnp.minimum(py2[q], gy2) - jnp.maximum(py1[q], gy1), 0.0)
                inter = wx * wy
                den = area_p[q] + gab - inter + 1e-9
                iou = inter / den
                upd = iou > bvs[q]
                nbv.append(jnp.where(upd, iou, bvs[q]))
                nbg.append(jnp.where(upd, g, bgs[q]))
            return tuple(nbv), tuple(nbg)

        bvs, bgs = lax.fori_loop(
            0, _G, gt_body,
            (tuple(jnp.full((16,), -1.0, _F32) for _ in range(_QB)),
             tuple(jnp.zeros((16,), _I32) for _ in range(_QB))),
        )
        for q in range(_QB):
            pidx = t * _CHUNK + (cb * _QB + q) * 16 + lane
            valid = pidx < _N
            fg = bvs[q] >= _HI
            kfg_r[sls[q]] = jnp.where(fg & valid, bvs[q], _NEG)
            kbg_r[sls[q]] = jnp.where((~fg) & valid, -bvs[q], _NEG)
            gti_r[sls[q]] = bgs[q]
        return 0

    lax.fori_loop(0, _NV // _QB, chunk_body, 0)

    tbase = t * _CHUNK

    def extract_top(scan_all, K, val_fn, gt_fn):
        nl = K // 16

        def body(k, carry):
            vlast, ilast, lv, li, lg = carry
            bv, bi = scan_all(vlast, ilast)
            vmax = jnp.max(bv)
            imin = jnp.min(jnp.where(bv == vmax, bi, _IMAX))
            slot = imin - tbase
            wm = lane == (k % 16)
            j = k // 16
            valv = val_fn(slot)
            iminv = _splat(imin, _I32)
            lv = tuple(jnp.where(wm & (j == jj), valv, lv[jj])
                       for jj in range(nl))
            li = tuple(jnp.where(wm & (j == jj), iminv, li[jj])
                       for jj in range(nl))
            if gt_fn is not None:
                gv = gt_fn(slot)
                lg = tuple(jnp.where(wm & (j == jj), gv, lg[jj])
                           for jj in range(nl))
            return valv, iminv, lv, li, lg

        zf = tuple(jnp.zeros((16,), _F32) for _ in range(nl))
        zi = tuple(jnp.zeros((16,), _I32) for _ in range(nl))
        zg = zi if gt_fn is not None else ()
        _, _, lv, li, lg = lax.fori_loop(
            0, K, body,
            (jnp.full((16,), jnp.inf, _F32), jnp.full((16,), -1, _I32),
             zf, zi, zg))
        return lv, li, lg

    def make_local_scan(keys_ref):
        ivp = tbase + lane

        def scan_all(vlast, ilast):
            def scan(v, inner):
                bv, bi = inner
                kv = keys_ref[pl.ds(v * 16, 16)]
                iv = ivp + v * 16
                elig = (kv < vlast) | ((kv == vlast) & (iv > ilast))
                m = elig & ((kv > bv) | ((kv == bv) & (iv < bi)))
                return (jnp.where(m, kv, bv), jnp.where(m, iv, bi))

            return lax.fori_loop(
                0, _NV, scan,
                (jnp.full((16,), _NEG, _F32), jnp.full((16,), _IMAX, _I32)))

        return scan_all

    lv, li, lg = extract_top(
        make_local_scan(kfg_r), _NFG,
        lambda slot: plsc.load_gather(kfg_r, [_splat(slot, _I32)]),
        lambda slot: plsc.load_gather(gti_r, [_splat(slot, _I32)]))
    for jj in range(_NFG // 16):
        sl = pl.ds(jj * 16, 16)
        fgv_l[sl] = lv[jj]
        fgi_l[sl] = li[jj]
        fgg_l[sl] = lg[jj]
    lvb, lib, _ = extract_top(
        make_local_scan(kbg_r), _NBG,
        lambda slot: plsc.load_gather(kbg_r, [_splat(slot, _I32)]), None)
    for jj in range(_NBG // 16):
        sl = pl.ds(jj * 16, 16)
        bgv_l[sl] = lvb[jj]
        bgi_l[sl] = lib[jj]

    pltpu.sync_copy(fgv_l, fgv_s.at[t])
    pltpu.sync_copy(fgi_l, fgi_s.at[t])
    pltpu.sync_copy(fgg_l, fgg_s.at[t])
    pltpu.sync_copy(bgv_l, bgv_s.at[t])
    pltpu.sync_copy(bgi_l, bgi_s.at[t])
    plsc.subcore_barrier()

    @pl.when(t == 0)
    def _phase2():
        pltpu.sync_copy(fgv_s, fgv_m)
        pltpu.sync_copy(fgi_s, fgi_m)
        pltpu.sync_copy(fgg_s, fgg_m)
        pltpu.sync_copy(bgv_s, bgv_m)
        pltpu.sync_copy(bgi_s, bgi_m)

        def merge_heads(vals_m, idx_m, gt_m, K):
            nl = K // 16

            def body(k, carry):
                head, lv, li, lg = carry
                cv = plsc.load_gather(vals_m, [lane, head])
                civ = plsc.load_gather(idx_m, [lane, head])
                vmax = jnp.max(cv)
                imin = jnp.min(jnp.where(cv == vmax, civ, _IMAX))
                wm0 = (cv == vmax) & (civ == imin)
                lw = jnp.min(jnp.where(wm0, lane, 16))
                hcol = jnp.min(jnp.where(wm0, head, _IMAX))
                valv = _bc2(vals_m, lw, hcol)
                iminv = _splat(imin, _I32)
                wm = lane == (k % 16)
                j = k // 16
                lv = tuple(jnp.where(wm & (j == jj), valv, lv[jj])
                           for jj in range(nl))
                li = tuple(jnp.where(wm & (j == jj), iminv, li[jj])
                           for jj in range(nl))
                if gt_m is not None:
                    gv = _bc2(gt_m, lw, hcol)
                    lg = tuple(jnp.where(wm & (j == jj), gv, lg[jj])
                               for jj in range(nl))
                head = head + wm0.astype(_I32)
                return head, lv, li, lg

            zf = tuple(jnp.zeros((16,), _F32) for _ in range(nl))
            zi = tuple(jnp.zeros((16,), _I32) for _ in range(nl))
            zg = zi if gt_m is not None else ()
            _, lv, li, lg = lax.fori_loop(
                0, K, body, (jnp.zeros((16,), _I32), zf, zi, zg))
            return lv, li, lg

        mlv, mli, mlg = merge_heads(fgv_m, fgi_m, fgg_m, _NFG)
        for jj in range(_NFG // 16):
            sl = pl.ds(jj * 16, 16)
            keep_r[sl] = mli[jj]
            keepg_r[sl] = mlg[jj]
        blv, bli, _ = merge_heads(bgv_m, bgi_m, None, _NBG)
        for jj in range(_NBG // 16):
            keep_r[pl.ds(_NFG + jj * 16, 16)] = bli[jj]

        pltpu.sync_copy(keep_r, keep_s)
        pltpu.sync_copy(keepg_r, keepg_s)

    plsc.subcore_barrier()

    pltpu.sync_copy(keep_s, keep_r)
    coord_refs = (px1_r, py1_r, px2_r, py2_r)
    for ch in range(8):
        idxv = keep_r[pl.ds(ch * 16, 16)]
        mine = (idxv >= tbase) & (idxv < tbase + _CHUNK)
        local = jnp.clip(idxv - tbase, 0, _CHUNK - 1)
        for c4 in range(4):
            vals = plsc.load_gather(coord_refs[c4], [local])
            pout4_r[c4, pl.ds(ch * 16, 16)] = jnp.where(mine, vals, 0.0)
    pltpu.sync_copy(pout4_r, pcon_s.at[t])
    plsc.subcore_barrier()

    @pl.when(t == 0)
    def _phase2c():
        pltpu.sync_copy(pcon_s, pcon_m)
        for c4 in range(4):
            for ch in range(8):
                sl = pl.ds(ch * 16, 16)
                acc = pcon_m[0, c4, sl]
                for ti in range(1, _NTILE):
                    acc = acc + pcon_m[ti, c4, sl]
                pout4_r[c4, sl] = acc
        for ch in range(32):
            p = ch * 16 + lane
            vals = plsc.load_gather(pout4_r, [p % 4, p // 4])
            pout_r[pl.ds(ch * 16, 16)] = vals
        pltpu.sync_copy(pout_r, p_o.at[b])

        for ch in range(2):
            gv = keepg_r[pl.ds(ch * 16, 16)]
            labv = plsc.load_gather(lab_r, [gv])
            lout_r[pl.ds(ch * 16, 16)] = labv
        zz = jnp.zeros((16,), _I32)
        for ch in range(6):
            lout_r[pl.ds(_NFG + ch * 16, 16)] = zz
        pltpu.sync_copy(lout_r, l_o.at[b])

        for ch in range(2):
            sl = pl.ds(ch * 16, 16)
            px1 = pout4_r[0, sl]
            py1 = pout4_r[1, sl]
            px2 = pout4_r[2, sl]
            py2 = pout4_r[3, sl]
            gv = keepg_r[sl]
            gx1 = plsc.load_gather(gx1_r, [gv])
            gy1 = plsc.load_gather(gy1_r, [gv])
            gx2 = plsc.load_gather(gx2_r, [gv])
            gy2 = plsc.load_gather(gy2_r, [gv])
            pw = px2 - px1
            ph = py2 - py1
            pxc = px1 + 0.5 * pw
            pyc = py1 + 0.5 * ph
            gw = gx2 - gx1
            gh = gy2 - gy1
            gxc = gx1 + 0.5 * gw
            gyc = gy1 + 0.5 * gh
            regs = (
                (gxc - pxc) / pw,
                (gyc - pyc) / ph,
                _log16(gw / pw),
                _log16(gh / ph),
            )
            for c4 in range(4):
                rout4_r[c4, sl] = regs[c4]
            ix1 = jnp.maximum(px1, gx1)
            iy1 = jnp.maximum(py1, gy1)
            ix2 = jnp.maximum(jnp.minimum(px2, gx2), ix1 + 1e-3)
            iy2 = jnp.maximum(jnp.minimum(py2, gy2), iy1 + 1e-3)
            inter_r[0, sl] = ix1
            inter_r[1, sl] = iy1
            inter_r[2, sl] = ix2
            inter_r[3, sl] = iy2
        for ch in range(8):
            p = ch * 16 + lane
            vals = plsc.load_gather(rout4_r, [p % 4, p // 4])
            rout_r[pl.ds(ch * 16, 16)] = vals
        pltpu.sync_copy(rout_r, r_o.at[b])
        pltpu.sync_copy(inter_r, inter_s)

    plsc.subcore_barrier()

    pltpu.sync_copy(inter_s, inter_r)
    pltpu.sync_copy(keepg_s, keepg_r)
    for so in range(2):
        s = t * 2 + so
        mgt = jnp.clip(plsc.load_gather(keepg_r, [_splat(s, _I32)]), 0, _G - 1)
        base_row = (b * _G + mgt) * _S
        x1 = _bc2(inter_r, 0, s)
        y1 = _bc2(inter_r, 1, s)
        x2 = _bc2(inter_r, 2, s)
        y2 = _bc2(inter_r, 3, s)
        bh = (y2 - y1) / float(_MASK)
        bw = (x2 - x1) / float(_MASK)
        for ch in range(2):
            iv = (lanef + (ch * 16)) + 0.5
            ys = y1 + iv * bh - 0.5
            ysc = jnp.clip(ys, 0.0, float(_S - 1))
            yf = ysc.astype(_I32)
            ycl = jnp.minimum(yf + 1, _S - 1)
            xs = x1 + iv * bw - 0.5
            xsc = jnp.clip(xs, 0.0, float(_S - 1))
            xf = xsc.astype(_I32)
            xcl = jnp.minimum(xf + 1, _S - 1)
            sl = pl.ds(ch * 16, 16)
            ridx_r[sl] = base_row + yf
            ridx_r[pl.ds(32 + ch * 16, 16)] = base_row + ycl
            x0_r[sl] = xf
            x1i_r[sl] = xcl
            wx_r[sl] = xsc - xf.astype(_F32)
            wy_r[sl] = ysc - yf.astype(_F32)
        pltpu.async_copy(mrow_hbm.at[ridx_r], mrows_r, sem).wait()

        def row_body(i, _):
            wyb = _bc1(wy_r, i)
            omy = 1.0 - wyb
            r0 = i
            r1 = 32 + i
            for ch in range(2):
                sl = pl.ds(ch * 16, 16)
                x0v = x0_r[sl]
                x1v = x1i_r[sl]
                wxv = wx_r[sl]
                omx = 1.0 - wxv
                v00 = plsc.load_gather(mrows_r, [_splat(r0, _I32), x0v])
                v01 = plsc.load_gather(mrows_r, [_splat(r0, _I32), x1v])
                v10 = plsc.load_gather(mrows_r, [_splat(r1, _I32), x0v])
                v11 = plsc.load_gather(mrows_r, [_splat(r1, _I32), x1v])
                mf = (v00 * omy * omx + v01 * omy * wxv
                      + v10 * wyb * omx + v11 * wyb * wxv)
                outv = (mf > 0.5).astype(_I32)
                mout_r[pl.ds(i * _MASK + ch * 16, 16)] = outv
            return 0

        lax.fori_loop(0, _MASK, row_body, 0)
        pltpu.sync_copy(mout_r, m_o.at[b, s])


def kernel(proposals, gt_labels, gt_bboxes, gt_masks):
    props_t = jnp.moveaxis(proposals, -1, 1)
    props_t = jnp.pad(props_t, ((0, 0), (0, 0), (0, _NPAD - _N)),
                      constant_values=1.0)
    gt_t = jnp.moveaxis(gt_bboxes, -1, 1)
    labels = gt_labels.astype(_I32)
    mrows = gt_masks.reshape(_B * _G * _S, _S)

    mesh = plsc.VectorSubcoreMesh(core_axis_name="c", subcore_axis_name="s")
    call = pl.kernel(
        _sc_body,
        out_type=(
            jax.ShapeDtypeStruct((_B, 512), _F32),
            jax.ShapeDtypeStruct((_B, 128), _I32),
            jax.ShapeDtypeStruct((_B, 128), _F32),
            jax.ShapeDtypeStruct((_B, 32, 800), _I32),
        ),
        mesh=mesh,
        compiler_params=pltpu.CompilerParams(needs_layout_passes=False),
        scratch_types=[
            pltpu.VMEM((_CHUNK,), _F32),
            pltpu.VMEM((_CHUNK,), _F32),
            pltpu.VMEM((_CHUNK,), _F32),
            pltpu.VMEM((_CHUNK,), _F32),
            pltpu.VMEM((_G,), _F32),
            pltpu.VMEM((_G,), _F32),
            pltpu.VMEM((_G,), _F32),
            pltpu.VMEM((_G,), _F32),
            pltpu.VMEM((_G,), _F32),
            pltpu.VMEM((_G,), _I32),
            pltpu.VMEM((_CHUNK,), _F32),
            pltpu.VMEM((_CHUNK,), _F32),
            pltpu.VMEM((_CHUNK,), _I32),
            pltpu.VMEM((128,), _F32),
            pltpu.VMEM((128,), _F32),
            pltpu.VMEM((128,), _I32),
            pltpu.VMEM((128,), _I32),
            pltpu.VMEM((128,), _I32),
            pltpu.VMEM_SHARED((_NTILE, 128), _F32),
            pltpu.VMEM_SHARED((_NTILE, 128), _I32),
            pltpu.VMEM_SHARED((_NTILE, 128), _I32),
            pltpu.VMEM_SHARED((_NTILE, 128), _F32),
            pltpu.VMEM_SHARED((_NTILE, 128), _I32),
            pltpu.VMEM_SHARED((128,), _I32),
            pltpu.VMEM_SHARED((_NFG,), _I32),
            pltpu.VMEM_SHARED((4, _NFG), _F32),
            pltpu.VMEM_SHARED((_NTILE, 4, 128), _F32),
            pltpu.VMEM((_NTILE, 128), _F32),
            pltpu.VMEM((_NTILE, 128), _I32),
            pltpu.VMEM((_NTILE, 128), _I32),
            pltpu.VMEM((_NTILE, 128), _F32),
            pltpu.VMEM((_NTILE, 128), _I32),
            pltpu.VMEM((_NTILE, 4, 128), _F32),
            pltpu.VMEM((4, 128), _F32),
            pltpu.VMEM((4, _NFG), _F32),
            pltpu.VMEM((128,), _I32),
            pltpu.VMEM((512,), _F32),
            pltpu.VMEM((128,), _I32),
            pltpu.VMEM((128,), _F32),
            pltpu.VMEM((_NFG,), _I32),
            pltpu.VMEM((4, _NFG), _F32),
            pltpu.VMEM((64,), _I32),
            pltpu.VMEM((64, _S), _F32),
            pltpu.VMEM((_NFG,), _I32),
            pltpu.VMEM((_NFG,), _I32),
            pltpu.VMEM((_NFG,), _F32),
            pltpu.VMEM((_NFG,), _F32),
            pltpu.VMEM((800,), _I32),
            pltpu.SemaphoreType.DMA,
        ],
    )
    p, l, r, mk = call(props_t, gt_t, labels, mrows)
    return (
        p.reshape(_B, 128, 4),
        l,
        r.reshape(_B, 32, 4),
        mk[:, :, : _MASK * _MASK].reshape(_B, 32, 28, 28),
    )


# speedup vs baseline: 2.8158x; 1.2922x over previous
import functools

import jax
import jax.numpy as jnp
from jax import lax
from jax.experimental import pallas as pl
from jax.experimental.pallas import tpu as pltpu
from jax.experimental.pallas import tpu_sc as plsc

_B, _N, _G, _S = 2, 20000, 64, 256
_HI = 0.5
_NFG, _NBG = 32, 96
_MASK = 28
_NTILE = 16
_CHUNK = 1280
_NV = _CHUNK // 16
_NPAD = _NTILE * _CHUNK

_F32 = jnp.float32
_I32 = jnp.int32
_NEG = float("-inf")
_IMAX = 2**31 - 1


def _lane():
    return jnp.arange(16, dtype=_I32)


def _splat(x, dtype):
    return jnp.broadcast_to(jnp.asarray(x, dtype), (16,))


def _bc1(ref, i):
    return plsc.load_gather(ref, [_splat(i, _I32)])


def _bc2(ref, i, j):
    return plsc.load_gather(ref, [_splat(i, _I32), _splat(j, _I32)])


def _log16(x):
    i = plsc.bitcast(x, _I32)
    e = ((i >> 23) & 0xFF) - 127
    m = plsc.bitcast((i & 0x7FFFFF) | 0x3F800000, _F32)
    big = m > 1.4142135
    m = jnp.where(big, m * 0.5, m)
    e = e + big.astype(_I32)
    s = (m - 1.0) / (m + 1.0)
    s2 = s * s
    p = s2 * (1.0 / 9.0)
    p = s2 * (p + 1.0 / 7.0)
    p = s2 * (p + 1.0 / 5.0)
    p = s2 * (p + 1.0 / 3.0)
    p = (p + 1.0) * (2.0 * s)
    return e.astype(_F32) * 0.6931471805599453 + p


def _sc_body(
    pt_hbm,
    gt_hbm,
    lab_hbm,
    mrow_hbm,
    p_o,
    l_o,
    r_o,
    m_o,
    px1_r, py1_r, px2_r, py2_r,
    gx1_r, gy1_r, gx2_r, gy2_r, gar_r,
    lab_r,
    kfg_r, kbg_r,
    gti_r,
    ckey_r, cidx_r,
    fgv_l, bgv_l,
    fgi_l, fgg_l, bgi_l,
    fgv_s, fgi_s, fgg_s,
    bgv_s, bgi_s,
    keep_s,
    keepg_s,
    inter_s,
    pcon_s,
    fgv_m, fgi_m, fgg_m,
    bgv_m, bgi_m,
    pcon_m,
    pout4_r,
    rout4_r,
    keep_r,
    pout_r,
    lout_r,
    rout_r,
    keepg_r,
    inter_r,
    ridx_r,
    mrows_r,
    x0_r, x1i_r,
    wx_r, wy_r,
    mout_r,
    sem,
):
    b = lax.axis_index("c")
    t = lax.axis_index("s")
    lane = _lane()
    lanef = lane.astype(_F32)

    pltpu.sync_copy(pt_hbm.at[b, 0, pl.ds(t * _CHUNK, _CHUNK)], px1_r)
    pltpu.sync_copy(pt_hbm.at[b, 1, pl.ds(t * _CHUNK, _CHUNK)], py1_r)
    pltpu.sync_copy(pt_hbm.at[b, 2, pl.ds(t * _CHUNK, _CHUNK)], px2_r)
    pltpu.sync_copy(pt_hbm.at[b, 3, pl.ds(t * _CHUNK, _CHUNK)], py2_r)
    pltpu.sync_copy(gt_hbm.at[b, 0], gx1_r)
    pltpu.sync_copy(gt_hbm.at[b, 1], gy1_r)
    pltpu.sync_copy(gt_hbm.at[b, 2], gx2_r)
    pltpu.sync_copy(gt_hbm.at[b, 3], gy2_r)
    pltpu.sync_copy(lab_hbm.at[b], lab_r)

    for c in range(_G // 16):
        sl = pl.ds(c * 16, 16)
        gar_r[sl] = (gx2_r[sl] - gx1_r[sl]) * (gy2_r[sl] - gy1_r[sl])

    _QB = 4

    def chunk_body(cb, _):
        sls = [pl.ds((cb * _QB + q) * 16, 16) for q in range(_QB)]
        px1 = [px1_r[s] for s in sls]
        py1 = [py1_r[s] for s in sls]
        px2 = [px2_r[s] for s in sls]
        py2 = [py2_r[s] for s in sls]
        area_p = [(px2[q] - px1[q]) * (py2[q] - py1[q]) for q in range(_QB)]

        def gt_body(g, carry):
            bvs, bgs = carry
            gx1 = _bc1(gx1_r, g)
            gy1 = _bc1(gy1_r, g)
            gx2 = _bc1(gx2_r, g)
            gy2 = _bc1(gy2_r, g)
            gab = _bc1(gar_r, g)
            nbv, nbg = [], []
            for q in range(_QB):
                wx = jnp.maximum(
                    jnp.minimum(px2[q], gx2) - jnp.maximum(px1[q], gx1), 0.0)
                wy = jnp.maximum(
                    jnp.minimum(py2[q], gy2) - jnp.maximum(py1[q], gy1), 0.0)
                inter = wx * wy
                den = area_p[q] + gab - inter + 1e-9
                iou = inter / den
                upd = iou > bvs[q]
                nbv.append(jnp.where(upd, iou, bvs[q]))
                nbg.append(jnp.where(upd, g, bgs[q]))
            return tuple(nbv), tuple(nbg)

        bvs, bgs = lax.fori_loop(
            0, _G, gt_body,
            (tuple(jnp.full((16,), -1.0, _F32) for _ in range(_QB)),
             tuple(jnp.zeros((16,), _I32) for _ in range(_QB))),
        )
        for q in range(_QB):
            pidx = t * _CHUNK + (cb * _QB + q) * 16 + lane
            valid = pidx < _N
            fg = bvs[q] >= _HI
            kfg_r[sls[q]] = jnp.where(fg & valid, bvs[q], _NEG)
            kbg_r[sls[q]] = jnp.where((~fg) & valid, -bvs[q], _NEG)
            gti_r[sls[q]] = bgs[q]
        return 0

    lax.fori_loop(0, _NV // _QB, chunk_body, 0)

    tbase = t * _CHUNK

    def extract_top(scan_all, K, val_fn, gt_fn):
        nl = K // 16

        def body(k, carry):
            vlast, ilast, lv, li, lg = carry
            bv, bi = scan_all(vlast, ilast)
            vmax = jnp.max(bv)
            imin = jnp.min(jnp.where(bv == vmax, bi, _IMAX))
            slot = imin - tbase
            wm = lane == (k % 16)
            j = k // 16
            valv = val_fn(slot)
            iminv = _splat(imin, _I32)
            lv = tuple(jnp.where(wm & (j == jj), valv, lv[jj])
                       for jj in range(nl))
            li = tuple(jnp.where(wm & (j == jj), iminv, li[jj])
                       for jj in range(nl))
            if gt_fn is not None:
                gv = gt_fn(slot)
                lg = tuple(jnp.where(wm & (j == jj), gv, lg[jj])
                           for jj in range(nl))
            return valv, iminv, lv, li, lg

        zf = tuple(jnp.zeros((16,), _F32) for _ in range(nl))
        zi = tuple(jnp.zeros((16,), _I32) for _ in range(nl))
        zg = zi if gt_fn is not None else ()
        _, _, lv, li, lg = lax.fori_loop(
            0, K, body,
            (jnp.full((16,), jnp.inf, _F32), jnp.full((16,), -1, _I32),
             zf, zi, zg))
        return lv, li, lg

    def topd_bound(keys_ref, D):
        def pre(v, s):
            c = keys_ref[pl.ds(v * 16, 16)]
            ns = []
            for j in range(D):
                hi = jnp.maximum(s[j], c)
                c = jnp.minimum(s[j], c)
                ns.append(hi)
            return tuple(ns)

        s = lax.fori_loop(
            0, _NV, pre, tuple(jnp.full((16,), _NEG, _F32) for _ in range(D)))
        return jnp.min(s[D - 1])

    def compact(keys_ref, bound):
        ivp = tbase + lane

        def comp(v, cnt):
            kv = keys_ref[pl.ds(v * 16, 16)]
            iv = ivp + v * 16
            msk = kv >= bound
            plsc.store_compressed(ckey_r.at[pl.ds(cnt, 16)], kv, mask=msk)
            plsc.store_compressed(cidx_r.at[pl.ds(cnt, 16)], iv, mask=msk)
            return cnt + plsc.all_reduce_population_count(msk)[0]

        cn = lax.fori_loop(0, _NV, comp, jnp.asarray(0, _I32))
        ckey_r[pl.ds(cn, 16)] = jnp.full((16,), _NEG, _F32)
        cidx_r[pl.ds(cn, 16)] = jnp.full((16,), _IMAX, _I32)
        return cn

    def extract_compacted(cn, K, want_gt):
        nl = K // 16
        nvc = (cn + 15) // 16

        def body(k, carry):
            vlast, ilast, lv, li, lg = carry

            def scan(v, inner):
                bv, bi, bs = inner
                kv = ckey_r[pl.ds(v * 16, 16)]
                iv = cidx_r[pl.ds(v * 16, 16)]
                slot = v * 16 + lane
                elig = (kv < vlast) | ((kv == vlast) & (iv > ilast))
                m = elig & ((kv > bv) | ((kv == bv) & (iv < bi)))
                return (
                    jnp.where(m, kv, bv),
                    jnp.where(m, iv, bi),
                    jnp.where(m, slot, bs),
                )

            bv, bi, bs = lax.fori_loop(
                0, nvc, scan,
                (jnp.full((16,), _NEG, _F32),
                 jnp.full((16,), _IMAX, _I32),
                 jnp.zeros((16,), _I32)))
            vmax = jnp.max(bv)
            imin = jnp.min(jnp.where(bv == vmax, bi, _IMAX))
            slot = jnp.min(jnp.where((bv == vmax) & (bi == imin), bs, _IMAX))
            wm = lane == (k % 16)
            j = k // 16
            valv = plsc.load_gather(ckey_r, [_splat(slot, _I32)])
            iminv = _splat(imin, _I32)
            lv = tuple(jnp.where(wm & (j == jj), valv, lv[jj])
                       for jj in range(nl))
            li = tuple(jnp.where(wm & (j == jj), iminv, li[jj])
                       for jj in range(nl))
            if want_gt:
                gv = plsc.load_gather(gti_r, [_splat(imin - tbase, _I32)])
                lg = tuple(jnp.where(wm & (j == jj), gv, lg[jj])
                           for jj in range(nl))
            return valv, iminv, lv, li, lg

        zf = tuple(jnp.zeros((16,), _F32) for _ in range(nl))
        zi = tuple(jnp.zeros((16,), _I32) for _ in range(nl))
        zg = zi if want_gt else ()
        _, _, lv, li, lg = lax.fori_loop(
            0, K, body,
            (jnp.full((16,), jnp.inf, _F32), jnp.full((16,), -1, _I32),
             zf, zi, zg))
        return lv, li, lg

    cn = compact(kfg_r, topd_bound(kfg_r, _NFG // 16))
    lv, li, lg = extract_compacted(cn, _NFG, True)
    for jj in range(_NFG // 16):
        sl = pl.ds(jj * 16, 16)
        fgv_l[sl] = lv[jj]
        fgi_l[sl] = li[jj]
        fgg_l[sl] = lg[jj]
    cn = compact(kbg_r, topd_bound(kbg_r, _NBG // 16))
    lvb, lib, _ = extract_compacted(cn, _NBG, False)
    for jj in range(_NBG // 16):
        sl = pl.ds(jj * 16, 16)
        bgv_l[sl] = lvb[jj]
        bgi_l[sl] = lib[jj]

    pltpu.sync_copy(fgv_l, fgv_s.at[t])
    pltpu.sync_copy(fgi_l, fgi_s.at[t])
    pltpu.sync_copy(fgg_l, fgg_s.at[t])
    pltpu.sync_copy(bgv_l, bgv_s.at[t])
    pltpu.sync_copy(bgi_l, bgi_s.at[t])
    plsc.subcore_barrier()

    @pl.when(t == 0)
    def _phase2():
        pltpu.sync_copy(fgv_s, fgv_m)
        pltpu.sync_copy(fgi_s, fgi_m)
        pltpu.sync_copy(fgg_s, fgg_m)
        pltpu.sync_copy(bgv_s, bgv_m)
        pltpu.sync_copy(bgi_s, bgi_m)

        def merge_heads(vals_m, idx_m, gt_m, K):
            nl = K // 16

            def body(k, carry):
                head, lv, li, lg = carry
                cv = plsc.load_gather(vals_m, [lane, head])
                civ = plsc.load_gather(idx_m, [lane, head])
                vmax = jnp.max(cv)
                imin = jnp.min(jnp.where(cv == vmax, civ, _IMAX))
                wm0 = (cv == vmax) & (civ == imin)
                lw = jnp.min(jnp.where(wm0, lane, 16))
                hcol = jnp.min(jnp.where(wm0, head, _IMAX))
                valv = _bc2(vals_m, lw, hcol)
                iminv = _splat(imin, _I32)
                wm = lane == (k % 16)
                j = k // 16
                lv = tuple(jnp.where(wm & (j == jj), valv, lv[jj])
                           for jj in range(nl))
                li = tuple(jnp.where(wm & (j == jj), iminv, li[jj])
                           for jj in range(nl))
                if gt_m is not None:
                    gv = _bc2(gt_m, lw, hcol)
                    lg = tuple(jnp.where(wm & (j == jj), gv, lg[jj])
                               for jj in range(nl))
                head = head + wm0.astype(_I32)
                return head, lv, li, lg

            zf = tuple(jnp.zeros((16,), _F32) for _ in range(nl))
            zi = tuple(jnp.zeros((16,), _I32) for _ in range(nl))
            zg = zi if gt_m is not None else ()
            _, lv, li, lg = lax.fori_loop(
                0, K, body, (jnp.zeros((16,), _I32), zf, zi, zg))
            return lv, li, lg

        mlv, mli, mlg = merge_heads(fgv_m, fgi_m, fgg_m, _NFG)
        for jj in range(_NFG // 16):
            sl = pl.ds(jj * 16, 16)
            keep_r[sl] = mli[jj]
            keepg_r[sl] = mlg[jj]
        blv, bli, _ = merge_heads(bgv_m, bgi_m, None, _NBG)
        for jj in range(_NBG // 16):
            keep_r[pl.ds(_NFG + jj * 16, 16)] = bli[jj]

        pltpu.sync_copy(keep_r, keep_s)
        pltpu.sync_copy(keepg_r, keepg_s)

    plsc.subcore_barrier()

    pltpu.sync_copy(keep_s, keep_r)
    coord_refs = (px1_r, py1_r, px2_r, py2_r)
    for ch in range(8):
        idxv = keep_r[pl.ds(ch * 16, 16)]
        mine = (idxv >= tbase) & (idxv < tbase + _CHUNK)
        local = jnp.clip(idxv - tbase, 0, _CHUNK - 1)
        for c4 in range(4):
            vals = plsc.load_gather(coord_refs[c4], [local])
            pout4_r[c4, pl.ds(ch * 16, 16)] = jnp.where(mine, vals, 0.0)
    pltpu.sync_copy(pout4_r, pcon_s.at[t])
    plsc.subcore_barrier()

    @pl.when(t == 0)
    def _phase2c():
        pltpu.sync_copy(pcon_s, pcon_m)
        for c4 in range(4):
            for ch in range(8):
                sl = pl.ds(ch * 16, 16)
                acc = pcon_m[0, c4, sl]
                for ti in range(1, _NTILE):
                    acc = acc + pcon_m[ti, c4, sl]
                pout4_r[c4, sl] = acc
        for ch in range(32):
            p = ch * 16 + lane
            vals = plsc.load_gather(pout4_r, [p % 4, p // 4])
            pout_r[pl.ds(ch * 16, 16)] = vals
        pltpu.sync_copy(pout_r, p_o.at[b])

        for ch in range(2):
            gv = keepg_r[pl.ds(ch * 16, 16)]
            labv = plsc.load_gather(lab_r, [gv])
            lout_r[pl.ds(ch * 16, 16)] = labv
        zz = jnp.zeros((16,), _I32)
        for ch in range(6):
            lout_r[pl.ds(_NFG + ch * 16, 16)] = zz
        pltpu.sync_copy(lout_r, l_o.at[b])

        for ch in range(2):
            sl = pl.ds(ch * 16, 16)
            px1 = pout4_r[0, sl]
            py1 = pout4_r[1, sl]
            px2 = pout4_r[2, sl]
            py2 = pout4_r[3, sl]
            gv = keepg_r[sl]
            gx1 = plsc.load_gather(gx1_r, [gv])
            gy1 = plsc.load_gather(gy1_r, [gv])
            gx2 = plsc.load_gather(gx2_r, [gv])
            gy2 = plsc.load_gather(gy2_r, [gv])
            pw = px2 - px1
            ph = py2 - py1
            pxc = px1 + 0.5 * pw
            pyc = py1 + 0.5 * ph
            gw = gx2 - gx1
            gh = gy2 - gy1
            gxc = gx1 + 0.5 * gw
            gyc = gy1 + 0.5 * gh
            regs = (
                (gxc - pxc) / pw,
                (gyc - pyc) / ph,
                _log16(gw / pw),
                _log16(gh / ph),
            )
            for c4 in range(4):
                rout4_r[c4, sl] = regs[c4]
            ix1 = jnp.maximum(px1, gx1)
            iy1 = jnp.maximum(py1, gy1)
            ix2 = jnp.maximum(jnp.minimum(px2, gx2), ix1 + 1e-3)
            iy2 = jnp.maximum(jnp.minimum(py2, gy2), iy1 + 1e-3)
            inter_r[0, sl] = ix1
            inter_r[1, sl] = iy1
            inter_r[2, sl] = ix2
            inter_r[3, sl] = iy2
        for ch in range(8):
            p = ch * 16 + lane
            vals = plsc.load_gather(rout4_r, [p % 4, p // 4])
            rout_r[pl.ds(ch * 16, 16)] = vals
        pltpu.sync_copy(rout_r, r_o.at[b])
        pltpu.sync_copy(inter_r, inter_s)

    plsc.subcore_barrier()

    pltpu.sync_copy(inter_s, inter_r)
    pltpu.sync_copy(keepg_s, keepg_r)
    for so in range(2):
        s = t * 2 + so
        mgt = jnp.clip(plsc.load_gather(keepg_r, [_splat(s, _I32)]), 0, _G - 1)
        base_row = (b * _G + mgt) * _S
        x1 = _bc2(inter_r, 0, s)
        y1 = _bc2(inter_r, 1, s)
        x2 = _bc2(inter_r, 2, s)
        y2 = _bc2(inter_r, 3, s)
        bh = (y2 - y1) / float(_MASK)
        bw = (x2 - x1) / float(_MASK)
        for ch in range(2):
            iv = (lanef + (ch * 16)) + 0.5
            ys = y1 + iv * bh - 0.5
            ysc = jnp.clip(ys, 0.0, float(_S - 1))
            yf = ysc.astype(_I32)
            ycl = jnp.minimum(yf + 1, _S - 1)
            xs = x1 + iv * bw - 0.5
            xsc = jnp.clip(xs, 0.0, float(_S - 1))
            xf = xsc.astype(_I32)
            xcl = jnp.minimum(xf + 1, _S - 1)
            sl = pl.ds(ch * 16, 16)
            ridx_r[sl] = base_row + yf
            ridx_r[pl.ds(32 + ch * 16, 16)] = base_row + ycl
            x0_r[sl] = xf
            x1i_r[sl] = xcl
            wx_r[sl] = xsc - xf.astype(_F32)
            wy_r[sl] = ysc - yf.astype(_F32)
        pltpu.async_copy(mrow_hbm.at[ridx_r], mrows_r, sem).wait()

        def row_body(i, _):
            wyb = _bc1(wy_r, i)
            omy = 1.0 - wyb
            r0 = i
            r1 = 32 + i
            for ch in range(2):
                sl = pl.ds(ch * 16, 16)
                x0v = x0_r[sl]
                x1v = x1i_r[sl]
                wxv = wx_r[sl]
                omx = 1.0 - wxv
                v00 = plsc.load_gather(mrows_r, [_splat(r0, _I32), x0v])
                v01 = plsc.load_gather(mrows_r, [_splat(r0, _I32), x1v])
                v10 = plsc.load_gather(mrows_r, [_splat(r1, _I32), x0v])
                v11 = plsc.load_gather(mrows_r, [_splat(r1, _I32), x1v])
                mf = (v00 * omy * omx + v01 * omy * wxv
                      + v10 * wyb * omx + v11 * wyb * wxv)
                outv = (mf > 0.5).astype(_I32)
                mout_r[pl.ds(i * _MASK + ch * 16, 16)] = outv
            return 0

        lax.fori_loop(0, _MASK, row_body, 0)
        pltpu.sync_copy(mout_r, m_o.at[b, s])


def kernel(proposals, gt_labels, gt_bboxes, gt_masks):
    props_t = jnp.moveaxis(proposals, -1, 1)
    props_t = jnp.pad(props_t, ((0, 0), (0, 0), (0, _NPAD - _N)),
                      constant_values=1.0)
    gt_t = jnp.moveaxis(gt_bboxes, -1, 1)
    labels = gt_labels.astype(_I32)
    mrows = gt_masks.reshape(_B * _G * _S, _S)

    mesh = plsc.VectorSubcoreMesh(core_axis_name="c", subcore_axis_name="s")
    call = pl.kernel(
        _sc_body,
        out_type=(
            jax.ShapeDtypeStruct((_B, 512), _F32),
            jax.ShapeDtypeStruct((_B, 128), _I32),
            jax.ShapeDtypeStruct((_B, 128), _F32),
            jax.ShapeDtypeStruct((_B, 32, 800), _I32),
        ),
        mesh=mesh,
        compiler_params=pltpu.CompilerParams(needs_layout_passes=False),
        scratch_types=[
            pltpu.VMEM((_CHUNK,), _F32),
            pltpu.VMEM((_CHUNK,), _F32),
            pltpu.VMEM((_CHUNK,), _F32),
            pltpu.VMEM((_CHUNK,), _F32),
            pltpu.VMEM((_G,), _F32),
            pltpu.VMEM((_G,), _F32),
            pltpu.VMEM((_G,), _F32),
            pltpu.VMEM((_G,), _F32),
            pltpu.VMEM((_G,), _F32),
            pltpu.VMEM((_G,), _I32),
            pltpu.VMEM((_CHUNK,), _F32),
            pltpu.VMEM((_CHUNK,), _F32),
            pltpu.VMEM((_CHUNK,), _I32),
            pltpu.VMEM((_CHUNK + 16,), _F32),
            pltpu.VMEM((_CHUNK + 16,), _I32),
            pltpu.VMEM((128,), _F32),
            pltpu.VMEM((128,), _F32),
            pltpu.VMEM((128,), _I32),
            pltpu.VMEM((128,), _I32),
            pltpu.VMEM((128,), _I32),
            pltpu.VMEM_SHARED((_NTILE, 128), _F32),
            pltpu.VMEM_SHARED((_NTILE, 128), _I32),
            pltpu.VMEM_SHARED((_NTILE, 128), _I32),
            pltpu.VMEM_SHARED((_NTILE, 128), _F32),
            pltpu.VMEM_SHARED((_NTILE, 128), _I32),
            pltpu.VMEM_SHARED((128,), _I32),
            pltpu.VMEM_SHARED((_NFG,), _I32),
            pltpu.VMEM_SHARED((4, _NFG), _F32),
            pltpu.VMEM_SHARED((_NTILE, 4, 128), _F32),
            pltpu.VMEM((_NTILE, 128), _F32),
            pltpu.VMEM((_NTILE, 128), _I32),
            pltpu.VMEM((_NTILE, 128), _I32),
            pltpu.VMEM((_NTILE, 128), _F32),
            pltpu.VMEM((_NTILE, 128), _I32),
            pltpu.VMEM((_NTILE, 4, 128), _F32),
            pltpu.VMEM((4, 128), _F32),
            pltpu.VMEM((4, _NFG), _F32),
            pltpu.VMEM((128,), _I32),
            pltpu.VMEM((512,), _F32),
            pltpu.VMEM((128,), _I32),
            pltpu.VMEM((128,), _F32),
            pltpu.VMEM((_NFG,), _I32),
            pltpu.VMEM((4, _NFG), _F32),
            pltpu.VMEM((64,), _I32),
            pltpu.VMEM((64, _S), _F32),
            pltpu.VMEM((_NFG,), _I32),
            pltpu.VMEM((_NFG,), _I32),
            pltpu.VMEM((_NFG,), _F32),
            pltpu.VMEM((_NFG,), _F32),
            pltpu.VMEM((800,), _I32),
            pltpu.SemaphoreType.DMA,
        ],
    )
    p, l, r, mk = call(props_t, gt_t, labels, mrows)
    return (
        p.reshape(_B, 128, 4),
        l,
        r.reshape(_B, 32, 4),
        mk[:, :, : _MASK * _MASK].reshape(_B, 32, 28, 28),
    )
